# Initial kernel scaffold; baseline (speedup 1.0000x reference)
import jax
import jax.numpy as jnp
from jax.experimental import pallas as pl


def kernel(preds, targets):
    raise NotImplementedError("write your pallas kernel here")


# speedup vs baseline: 2.5563x; 2.5563x over previous
import functools

import jax
import jax.numpy as jnp
from jax import lax
from jax.experimental import pallas as pl
from jax.experimental.pallas import tpu as pltpu
from jax.experimental.pallas import tpu_sc as plsc

NC = 2
NS = 16
L = 16
NB = 8
NPIX = 512 * 512
BPC = NB // NC
P = NPIX // NS
NIT = P // L

_DELTA_AGG = 0.5
_DELTA_DIS = 3.0

_PAIRS = [(a, b) for a in range(1, 8) for b in range(a + 1, 8)]
_PAIRS = _PAIRS + [(1, 1)] * (32 - len(_PAIRS))
_PAIR_TAB = [[[p[0] for p in _PAIRS[k * 16:(k + 1) * 16]],
              [p[1] for p in _PAIRS[k * 16:(k + 1) * 16]]] for k in range(2)]


def _recip16(x):
    i = jnp.int32(0x7EF477D5) - plsc.bitcast(x, jnp.int32)
    y = plsc.bitcast(i, jnp.float32)
    for _ in range(3):
        y = y * (2.0 - x * y)
    return y


def _sqrt16(s):
    i = plsc.bitcast(s, jnp.int32)
    i = jnp.int32(0x5F3759DF) - (i >> 1)
    y = plsc.bitcast(i, jnp.float32)
    for _ in range(3):
        y = y * (1.5 - 0.5 * s * y * y)
    return s * y


def _log16(x):
    b = plsc.bitcast(x, jnp.int32)
    e = ((b >> 23) - 127).astype(jnp.float32)
    m = plsc.bitcast((b & jnp.int32(0x007FFFFF)) | jnp.int32(0x3F800000),
                     jnp.float32)
    big = m > 1.4142135
    m = jnp.where(big, m * 0.5, m)
    e = e + jnp.where(big, 1.0, 0.0)
    z = (m - 1.0) * _recip16(m + 1.0)
    z2 = z * z
    logm = 2.0 * z * (1.0 + z2 * (1.0 / 3.0 + z2 * (0.2 + z2 * (1.0 / 7.0
                                                                + z2 / 9.0))))
    return e * 0.6931471805599453 + logm


def _sc_body(preds_hbm, tgt_hbm, pairs_hbm, out_hbm,
             sv_v, tt_v, tk_v, tab_v, tmp_v, acc_v, gtab_v,
             stat_v, outbuf_v, pairs_v, shared_v, sem):
    cid = lax.axis_index("c")
    sid = lax.axis_index("s")
    lane = lax.iota(jnp.int32, L)
    zeros = jnp.zeros((L,), jnp.float32)
    ones = jnp.full((L,), 1.0, jnp.float32)

    pltpu.sync_copy(pairs_hbm, pairs_v)

    @pl.when(sid == 0)
    def _():
        for r in range(2):
            outbuf_v[r] = zeros

    def reduce_tables():
        pltpu.sync_copy(shared_v.at[0], acc_v)

        def rbody(s, _):
            pltpu.sync_copy(shared_v.at[s], tmp_v)
            for r in range(8):
                for j in range(128 // L):
                    ds = pl.ds(j * L, L)
                    acc_v[r, ds] = acc_v[r, ds] + tmp_v[r, ds]
            return 0

        lax.fori_loop(1, NS, rbody, 0)

    def batch_body(bl, _):
        b = cid * BPC + bl
        base = sid * P

        cps = []
        for ch in range(4):
            cps.append(pltpu.async_copy(
                preds_hbm.at[b, 2 + ch, pl.ds(base, P)], sv_v.at[ch], sem))
        cps.append(pltpu.async_copy(tgt_hbm.at[b, 0, pl.ds(base, P)], tt_v,
                                    sem))
        cps.append(pltpu.async_copy(tgt_hbm.at[b, 1, pl.ds(base, P)], tk_v,
                                    sem))
        for cp in cps:
            cp.wait()

        for r in range(8):
            for j in range(128 // L):
                tab_v[r, pl.ds(j * L, L)] = zeros

        def body1(i, _):
            off = i * L
            idx = (tk_v[pl.ds(off, L)] << 4) | lane
            plsc.addupdate_scatter(tab_v.at[4], [idx], ones)
            for ch in range(4):
                plsc.addupdate_scatter(tab_v.at[ch], [idx],
                                       sv_v[ch, pl.ds(off, L)])
            return 0

        lax.fori_loop(0, NIT, body1, 0)

        pltpu.sync_copy(tab_v, shared_v.at[sid])
        plsc.subcore_barrier()
        reduce_tables()
        ckvec = zeros
        gv = [zeros, zeros, zeros, zeros]
        for i in range(8):
            sel = lane == i
            ckvec = jnp.where(sel, jnp.sum(acc_v[4, pl.ds(i * L, L)]), ckvec)
            for ch in range(4):
                gv[ch] = jnp.where(sel, jnp.sum(acc_v[ch, pl.ds(i * L, L)]),
                                   gv[ch])
        stat_v[0] = ckvec
        inv = _recip16(jnp.maximum(ckvec, 1.0))
        for ch in range(4):
            gtab_v[ch] = gv[ch] * inv
        plsc.subcore_barrier()

        def body2(i, _):
            off = i * L
            tt16 = tt_v[pl.ds(off, L)]
            s = zeros
            for ch in range(4):
                d = sv_v[ch, pl.ds(off, L)] - plsc.load_gather(gtab_v.at[ch],
                                                               [tt16])
                s = s + d * d
            d = _sqrt16(jnp.maximum(s, 0.25)) - _DELTA_AGG
            lg = _log16(d * d + 1.0)
            idx = (tt16 << 4) | lane
            plsc.addupdate_scatter(tab_v.at[5], [idx], lg)
            plsc.addupdate_scatter(tab_v.at[6], [idx], ones)
            return 0

        lax.fori_loop(0, NIT, body2, 0)

        pltpu.sync_copy(tab_v, shared_v.at[sid])
        plsc.subcore_barrier()

        @pl.when(sid == 0)
        def _():
            reduce_tables()
            lsum = zeros
            ctv = zeros
            for i in range(1, 8):
                sel = lane == i
                lsum = jnp.where(sel, jnp.sum(acc_v[5, pl.ds(i * L, L)]),
                                 lsum)
                ctv = jnp.where(sel, jnp.sum(acc_v[6, pl.ds(i * L, L)]), ctv)
            ckv = stat_v[0]
            lanemask = (lane >= 1) & (lane < 8)
            validf = jnp.where((ctv > 0.0) & (ckv > 0.0) & lanemask, 1.0, 0.0)
            stat_v[3] = validf
            m = zeros + jnp.sum(validf)
            inst = lsum * _recip16(jnp.maximum(ctv, 1.0))
            aggsum = jnp.sum(validf * inst)
            agg_b = jnp.where(m > 0.0,
                              aggsum * _recip16(jnp.maximum(m, 1.0)), 0.0)
            dsum = 0.0
            for k in range(2):
                pa = pairs_v[k, 0]
                pb = pairs_v[k, 1]
                pm = jnp.where(pa < pb, 1.0, 0.0)
                s = zeros
                for ch in range(4):
                    dg = (plsc.load_gather(gtab_v.at[ch], [pa])
                          - plsc.load_gather(gtab_v.at[ch], [pb]))
                    s = s + dg * dg
                nrm = _sqrt16(jnp.maximum(s, 1e-12))
                dd = jnp.maximum(_DELTA_DIS - nrm, 0.0)
                va = plsc.load_gather(stat_v.at[3], [pa])
                vb = plsc.load_gather(stat_v.at[3], [pb])
                dsum = dsum + jnp.sum(_log16(dd * dd + 1.0) * va * vb * pm)
            denom = jnp.maximum(m * (m - 1.0), 1.0)
            dis_b = jnp.where(m > 1.0, dsum * _recip16(denom), 0.0)
            outbuf_v[0] = jnp.where(lane == bl, agg_b, outbuf_v[0])
            outbuf_v[1] = jnp.where(lane == bl, dis_b, outbuf_v[1])
        plsc.subcore_barrier()
        return 0

    lax.fori_loop(0, BPC, batch_body, 0)

    @pl.when(sid == 0)
    def _():
        pltpu.sync_copy(outbuf_v, out_hbm.at[cid])


@jax.jit
def kernel(preds, targets):
    preds_r = preds.reshape(NB, 6, NPIX)
    tgt_r = targets.reshape(NB, 2, NPIX)
    mesh = plsc.VectorSubcoreMesh(core_axis_name="c", subcore_axis_name="s",
                                  num_cores=NC, num_subcores=NS)
    out = pl.kernel(
        _sc_body,
        out_type=jax.ShapeDtypeStruct((NC, 2, L), jnp.float32),
        mesh=mesh,
        compiler_params=pltpu.CompilerParams(needs_layout_passes=False),
        scratch_types=[
            pltpu.VMEM((4, P), jnp.float32),
            pltpu.VMEM((P,), jnp.int32),
            pltpu.VMEM((P,), jnp.int32),
            pltpu.VMEM((8, 128), jnp.float32),
            pltpu.VMEM((8, 128), jnp.float32),
            pltpu.VMEM((8, 128), jnp.float32),
            pltpu.VMEM((4, L), jnp.float32),
            pltpu.VMEM((4, L), jnp.float32),
            pltpu.VMEM((2, L), jnp.float32),
            pltpu.VMEM((2, 2, L), jnp.int32),
            pltpu.VMEM_SHARED((NS, 8, 128), jnp.float32),
            pltpu.SemaphoreType.DMA,
        ],
    )(preds_r, tgt_r, jnp.asarray(_PAIR_TAB, jnp.int32))
    agg = jnp.concatenate([out[0, 0, :BPC], out[1, 0, :BPC]])
    dis = jnp.concatenate([out[0, 1, :BPC], out[1, 1, :BPC]])
    return agg, dis


# speedup vs baseline: 5.7725x; 2.2581x over previous
import functools

import jax
import jax.numpy as jnp
from jax import lax
from jax.experimental import pallas as pl
from jax.experimental.pallas import tpu as pltpu
from jax.experimental.pallas import tpu_sc as plsc

NC = 2
NS = 16
L = 16
NB = 8
NPIX = 512 * 512
BPC = NB // NC
P = NPIX // NS
NIT = P // L

_DELTA_AGG = 0.5
_DELTA_DIS = 3.0

_PAIRS = [(a, b) for a in range(1, 8) for b in range(a + 1, 8)]
_PAIRS = _PAIRS + [(1, 1)] * (32 - len(_PAIRS))
_PAIR_TAB = [[[p[0] for p in _PAIRS[k * 16:(k + 1) * 16]],
              [p[1] for p in _PAIRS[k * 16:(k + 1) * 16]]] for k in range(2)]


def _recip16(x):
    i = jnp.int32(0x7EF477D5) - plsc.bitcast(x, jnp.int32)
    y = plsc.bitcast(i, jnp.float32)
    for _ in range(2):
        y = y * (2.0 - x * y)
    return y


def _sqrt16(s):
    i = plsc.bitcast(s, jnp.int32)
    i = jnp.int32(0x5F3759DF) - (i >> 1)
    y = plsc.bitcast(i, jnp.float32)
    for _ in range(2):
        y = y * (1.5 - 0.5 * s * y * y)
    return s * y


_LOGC = (-2.242481818574011, 4.911042808768637, -5.126667255636585,
         3.9326333882277567, -2.020202093852692, 0.6590148821973966,
         -0.12345843186233507, 0.010119082927734654)


def _log16(x):
    b = plsc.bitcast(x, jnp.int32)
    e = ((b >> 23) - 127).astype(jnp.float32)
    m = plsc.bitcast((b & jnp.int32(0x007FFFFF)) | jnp.int32(0x3F800000),
                     jnp.float32)
    acc = jnp.full((L,), _LOGC[7], jnp.float32)
    for c in _LOGC[6::-1]:
        acc = acc * m + c
    return e * 0.6931471805599453 + acc


def _sc_body(preds_hbm, tgt_hbm, pairs_hbm, out_hbm,
             sv_v, tt_v, tk_v, tab_v, tmp_v, acc_v, gtab_v,
             stat_v, outbuf_v, pairs_v, shared_v, sem):
    cid = lax.axis_index("c")
    sid = lax.axis_index("s")
    lane = lax.iota(jnp.int32, L)
    zeros = jnp.zeros((L,), jnp.float32)
    ones = jnp.full((L,), 1.0, jnp.float32)

    pltpu.sync_copy(pairs_hbm, pairs_v)

    @pl.when(sid == 0)
    def _():
        for r in range(2):
            outbuf_v[r] = zeros

    def reduce_tables(r0, nr):
        pltpu.sync_copy(shared_v.at[0, pl.ds(r0, nr)], acc_v.at[pl.ds(r0, nr)])

        def rbody(s, _):
            pltpu.sync_copy(shared_v.at[s, pl.ds(r0, nr)],
                            tmp_v.at[pl.ds(0, nr)])
            for r in range(nr):
                for j in range(128 // L):
                    ds = pl.ds(j * L, L)
                    acc_v[r0 + r, ds] = acc_v[r0 + r, ds] + tmp_v[r, ds]
            return 0

        lax.fori_loop(1, NS, rbody, 0)

    def batch_body(bl, _):
        b = cid * BPC + bl
        base = sid * P

        cps = []
        for ch in range(4):
            cps.append(pltpu.async_copy(
                preds_hbm.at[b, 2 + ch, pl.ds(base, P)], sv_v.at[ch], sem))
        cps.append(pltpu.async_copy(tgt_hbm.at[b, 0, pl.ds(base, P)], tt_v,
                                    sem))
        cps.append(pltpu.async_copy(tgt_hbm.at[b, 1, pl.ds(base, P)], tk_v,
                                    sem))
        for cp in cps:
            cp.wait()

        for r in range(8):
            for j in range(128 // L):
                tab_v[r, pl.ds(j * L, L)] = zeros

        @plsc.parallel_loop(0, NIT, 1, unroll=8)
        def _(i):
            off = i * L
            idx = (tk_v[pl.ds(off, L)] << 4) | lane
            plsc.addupdate_scatter(tab_v.at[4], [idx], ones)
            for ch in range(4):
                plsc.addupdate_scatter(tab_v.at[ch], [idx],
                                       sv_v[ch, pl.ds(off, L)])

        pltpu.sync_copy(tab_v.at[pl.ds(0, 5)], shared_v.at[sid, pl.ds(0, 5)])
        plsc.subcore_barrier()
        reduce_tables(0, 5)
        ckvec = zeros
        gv = [zeros, zeros, zeros, zeros]
        for i in range(8):
            sel = lane == i
            ckvec = jnp.where(sel, jnp.sum(acc_v[4, pl.ds(i * L, L)]), ckvec)
            for ch in range(4):
                gv[ch] = jnp.where(sel, jnp.sum(acc_v[ch, pl.ds(i * L, L)]),
                                   gv[ch])
        stat_v[0] = ckvec
        inv = _recip16(jnp.maximum(ckvec, 1.0))
        for ch in range(4):
            gtab_v[ch] = gv[ch] * inv
        plsc.subcore_barrier()

        @plsc.parallel_loop(0, NIT, 1, unroll=6)
        def _(i):
            off = i * L
            tt16 = tt_v[pl.ds(off, L)]
            s = zeros
            for ch in range(4):
                d = sv_v[ch, pl.ds(off, L)] - plsc.load_gather(gtab_v.at[ch],
                                                               [tt16])
                s = s + d * d
            s = jnp.maximum(s, 0.25)
            lg = _log16(s - _sqrt16(s) + 1.25)
            idx = (tt16 << 4) | lane
            plsc.addupdate_scatter(tab_v.at[5], [idx], lg)
            plsc.addupdate_scatter(tab_v.at[6], [idx], ones)

        pltpu.sync_copy(tab_v.at[pl.ds(5, 2)], shared_v.at[sid, pl.ds(5, 2)])
        plsc.subcore_barrier()

        @pl.when(sid == 0)
        def _():
            reduce_tables(5, 2)
            lsum = zeros
            ctv = zeros
            for i in range(1, 8):
                sel = lane == i
                lsum = jnp.where(sel, jnp.sum(acc_v[5, pl.ds(i * L, L)]),
                                 lsum)
                ctv = jnp.where(sel, jnp.sum(acc_v[6, pl.ds(i * L, L)]), ctv)
            ckv = stat_v[0]
            lanemask = (lane >= 1) & (lane < 8)
            validf = jnp.where((ctv > 0.0) & (ckv > 0.0) & lanemask, 1.0, 0.0)
            stat_v[3] = validf
            m = zeros + jnp.sum(validf)
            inst = lsum * _recip16(jnp.maximum(ctv, 1.0))
            aggsum = jnp.sum(validf * inst)
            agg_b = jnp.where(m > 0.0,
                              aggsum * _recip16(jnp.maximum(m, 1.0)), 0.0)
            dsum = 0.0
            for k in range(2):
                pa = pairs_v[k, 0]
                pb = pairs_v[k, 1]
                pm = jnp.where(pa < pb, 1.0, 0.0)
                s = zeros
                for ch in range(4):
                    dg = (plsc.load_gather(gtab_v.at[ch], [pa])
                          - plsc.load_gather(gtab_v.at[ch], [pb]))
                    s = s + dg * dg
                nrm = _sqrt16(jnp.maximum(s, 1e-12))
                dd = jnp.maximum(_DELTA_DIS - nrm, 0.0)
                va = plsc.load_gather(stat_v.at[3], [pa])
                vb = plsc.load_gather(stat_v.at[3], [pb])
                dsum = dsum + jnp.sum(_log16(dd * dd + 1.0) * va * vb * pm)
            denom = jnp.maximum(m * (m - 1.0), 1.0)
            dis_b = jnp.where(m > 1.0, dsum * _recip16(denom), 0.0)
            outbuf_v[0] = jnp.where(lane == bl, agg_b, outbuf_v[0])
            outbuf_v[1] = jnp.where(lane == bl, dis_b, outbuf_v[1])
        plsc.subcore_barrier()
        return 0

    lax.fori_loop(0, BPC, batch_body, 0)

    @pl.when(sid == 0)
    def _():
        pltpu.sync_copy(outbuf_v, out_hbm.at[cid])


@jax.jit
def kernel(preds, targets):
    preds_r = preds.reshape(NB, 6, NPIX)
    tgt_r = targets.reshape(NB, 2, NPIX)
    mesh = plsc.VectorSubcoreMesh(core_axis_name="c", subcore_axis_name="s",
                                  num_cores=NC, num_subcores=NS)
    out = pl.kernel(
        _sc_body,
        out_type=jax.ShapeDtypeStruct((NC, 2, L), jnp.float32),
        mesh=mesh,
        compiler_params=pltpu.CompilerParams(needs_layout_passes=False),
        scratch_types=[
            pltpu.VMEM((4, P), jnp.float32),
            pltpu.VMEM((P,), jnp.int32),
            pltpu.VMEM((P,), jnp.int32),
            pltpu.VMEM((8, 128), jnp.float32),
            pltpu.VMEM((8, 128), jnp.float32),
            pltpu.VMEM((8, 128), jnp.float32),
            pltpu.VMEM((4, L), jnp.float32),
            pltpu.VMEM((4, L), jnp.float32),
            pltpu.VMEM((2, L), jnp.float32),
            pltpu.VMEM((2, 2, L), jnp.int32),
            pltpu.VMEM_SHARED((NS, 8, 128), jnp.float32),
            pltpu.SemaphoreType.DMA,
        ],
    )(preds_r, tgt_r, jnp.asarray(_PAIR_TAB, jnp.int32))
    agg = jnp.concatenate([out[0, 0, :BPC], out[1, 0, :BPC]])
    dis = jnp.concatenate([out[0, 1, :BPC], out[1, 1, :BPC]])
    return agg, dis


# speedup vs baseline: 6.0284x; 1.0443x over previous
import functools

import jax
import jax.numpy as jnp
from jax import lax
from jax.experimental import pallas as pl
from jax.experimental.pallas import tpu as pltpu
from jax.experimental.pallas import tpu_sc as plsc

NC = 2
NS = 16
L = 16
NB = 8
NPIX = 512 * 512
BPC = NB // NC
P = NPIX // NS
NIT = P // L

_DELTA_AGG = 0.5
_DELTA_DIS = 3.0

_PAIRS = [(a, b) for a in range(1, 8) for b in range(a + 1, 8)]
_PAIRS = _PAIRS + [(1, 1)] * (32 - len(_PAIRS))
_PAIR_TAB = [[[p[0] for p in _PAIRS[k * 16:(k + 1) * 16]],
              [p[1] for p in _PAIRS[k * 16:(k + 1) * 16]]] for k in range(2)]


def _recip16(x):
    i = jnp.int32(0x7EF477D5) - plsc.bitcast(x, jnp.int32)
    y = plsc.bitcast(i, jnp.float32)
    for _ in range(2):
        y = y * (2.0 - x * y)
    return y


def _sqrt16(s, iters=2):
    i = plsc.bitcast(s, jnp.int32)
    i = jnp.int32(0x5F3759DF) - (i >> 1)
    y = plsc.bitcast(i, jnp.float32)
    for _ in range(iters):
        y = y * (1.5 - 0.5 * s * y * y)
    return s * y


_LOGC = (-2.242481818574011, 4.911042808768637, -5.126667255636585,
         3.9326333882277567, -2.020202093852692, 0.6590148821973966,
         -0.12345843186233507, 0.010119082927734654)


def _log16(x):
    b = plsc.bitcast(x, jnp.int32)
    e = ((b >> 23) - 127).astype(jnp.float32)
    m = plsc.bitcast((b & jnp.int32(0x007FFFFF)) | jnp.int32(0x3F800000),
                     jnp.float32)
    acc = jnp.full((L,), _LOGC[7], jnp.float32)
    for c in _LOGC[6::-1]:
        acc = acc * m + c
    return e * 0.6931471805599453 + acc


def _sc_body(preds_hbm, tgt_hbm, pairs_hbm, out_hbm,
             sv_v, tt_v, tk_v, tab_v, tmp_v, acc_v, gtab_v,
             stat_v, outbuf_v, pairs_v, shared_v, sem):
    cid = lax.axis_index("c")
    sid = lax.axis_index("s")
    lane = lax.iota(jnp.int32, L)
    zeros = jnp.zeros((L,), jnp.float32)
    ones = jnp.full((L,), 1.0, jnp.float32)

    pltpu.sync_copy(pairs_hbm, pairs_v)

    @pl.when(sid == 0)
    def _():
        for r in range(2):
            outbuf_v[r] = zeros

    def reduce_tables(r0, nr):
        pltpu.sync_copy(shared_v.at[0, pl.ds(r0, nr)], acc_v.at[pl.ds(r0, nr)])

        def rbody(s, _):
            pltpu.sync_copy(shared_v.at[s, pl.ds(r0, nr)],
                            tmp_v.at[pl.ds(0, nr)])
            for r in range(nr):
                for j in range(128 // L):
                    ds = pl.ds(j * L, L)
                    acc_v[r0 + r, ds] = acc_v[r0 + r, ds] + tmp_v[r, ds]
            return 0

        lax.fori_loop(1, NS, rbody, 0)

    base = sid * P

    def strip_copies(b, make):
        mk = pltpu.make_async_copy if make else pltpu.async_copy
        cps = []
        for ch in range(4):
            cps.append(mk(preds_hbm.at[b, 2 + ch, pl.ds(base, P)],
                          sv_v.at[ch], sem))
        cps.append(mk(tgt_hbm.at[b, 0, pl.ds(base, P)], tt_v, sem))
        cps.append(mk(tgt_hbm.at[b, 1, pl.ds(base, P)], tk_v, sem))
        return cps

    strip_copies(cid * BPC, make=False)

    def batch_body(bl, _):
        b = cid * BPC + bl

        for cp in strip_copies(b, make=True):
            cp.wait()

        for r in range(8):
            for j in range(128 // L):
                tab_v[r, pl.ds(j * L, L)] = zeros

        @plsc.parallel_loop(0, NIT, 1, unroll=8)
        def _(i):
            off = i * L
            idx = (tk_v[pl.ds(off, L)] << 4) | lane
            plsc.addupdate_scatter(tab_v.at[4], [idx], ones)
            for ch in range(4):
                plsc.addupdate_scatter(tab_v.at[ch], [idx],
                                       sv_v[ch, pl.ds(off, L)])

        pltpu.sync_copy(tab_v.at[pl.ds(0, 5)], shared_v.at[sid, pl.ds(0, 5)])
        plsc.subcore_barrier()
        reduce_tables(0, 5)
        ckvec = zeros
        gv = [zeros, zeros, zeros, zeros]
        for i in range(8):
            sel = lane == i
            ckvec = jnp.where(sel, jnp.sum(acc_v[4, pl.ds(i * L, L)]), ckvec)
            for ch in range(4):
                gv[ch] = jnp.where(sel, jnp.sum(acc_v[ch, pl.ds(i * L, L)]),
                                   gv[ch])
        stat_v[0] = ckvec
        inv = _recip16(jnp.maximum(ckvec, 1.0))
        for ch in range(4):
            gtab_v[ch] = gv[ch] * inv
        plsc.subcore_barrier()

        @plsc.parallel_loop(0, NIT, 1, unroll=6)
        def _(i):
            off = i * L
            tt16 = tt_v[pl.ds(off, L)]
            s = zeros
            for ch in range(4):
                d = sv_v[ch, pl.ds(off, L)] - plsc.load_gather(gtab_v.at[ch],
                                                               [tt16])
                s = s + d * d
            s = jnp.maximum(s, 0.25)
            lg = _log16(s - _sqrt16(s, iters=1) + 1.25)
            idx = (tt16 << 4) | lane
            plsc.addupdate_scatter(tab_v.at[5], [idx], lg)
            plsc.addupdate_scatter(tab_v.at[6], [idx], ones)

        @pl.when(bl < BPC - 1)
        def _():
            strip_copies(b + 1, make=False)

        pltpu.sync_copy(tab_v.at[pl.ds(5, 2)], shared_v.at[sid, pl.ds(5, 2)])
        plsc.subcore_barrier()

        @pl.when(sid == 0)
        def _():
            reduce_tables(5, 2)
            lsum = zeros
            ctv = zeros
            for i in range(1, 8):
                sel = lane == i
                lsum = jnp.where(sel, jnp.sum(acc_v[5, pl.ds(i * L, L)]),
                                 lsum)
                ctv = jnp.where(sel, jnp.sum(acc_v[6, pl.ds(i * L, L)]), ctv)
            ckv = stat_v[0]
            lanemask = (lane >= 1) & (lane < 8)
            validf = jnp.where((ctv > 0.0) & (ckv > 0.0) & lanemask, 1.0, 0.0)
            stat_v[3] = validf
            m = zeros + jnp.sum(validf)
            inst = lsum * _recip16(jnp.maximum(ctv, 1.0))
            aggsum = jnp.sum(validf * inst)
            agg_b = jnp.where(m > 0.0,
                              aggsum * _recip16(jnp.maximum(m, 1.0)), 0.0)
            dsum = 0.0
            for k in range(2):
                pa = pairs_v[k, 0]
                pb = pairs_v[k, 1]
                pm = jnp.where(pa < pb, 1.0, 0.0)
                s = zeros
                for ch in range(4):
                    dg = (plsc.load_gather(gtab_v.at[ch], [pa])
                          - plsc.load_gather(gtab_v.at[ch], [pb]))
                    s = s + dg * dg
                nrm = _sqrt16(jnp.maximum(s, 1e-12))
                dd = jnp.maximum(_DELTA_DIS - nrm, 0.0)
                va = plsc.load_gather(stat_v.at[3], [pa])
                vb = plsc.load_gather(stat_v.at[3], [pb])
                dsum = dsum + jnp.sum(_log16(dd * dd + 1.0) * va * vb * pm)
            denom = jnp.maximum(m * (m - 1.0), 1.0)
            dis_b = jnp.where(m > 1.0, dsum * _recip16(denom), 0.0)
            outbuf_v[0] = jnp.where(lane == bl, agg_b, outbuf_v[0])
            outbuf_v[1] = jnp.where(lane == bl, dis_b, outbuf_v[1])
        plsc.subcore_barrier()
        return 0

    lax.fori_loop(0, BPC, batch_body, 0)

    @pl.when(sid == 0)
    def _():
        pltpu.sync_copy(outbuf_v, out_hbm.at[cid])


@jax.jit
def kernel(preds, targets):
    preds_r = preds.reshape(NB, 6, NPIX)
    tgt_r = targets.reshape(NB, 2, NPIX)
    mesh = plsc.VectorSubcoreMesh(core_axis_name="c", subcore_axis_name="s",
                                  num_cores=NC, num_subcores=NS)
    out = pl.kernel(
        _sc_body,
        out_type=jax.ShapeDtypeStruct((NC, 2, L), jnp.float32),
        mesh=mesh,
        compiler_params=pltpu.CompilerParams(needs_layout_passes=False),
        scratch_types=[
            pltpu.VMEM((4, P), jnp.float32),
            pltpu.VMEM((P,), jnp.int32),
            pltpu.VMEM((P,), jnp.int32),
            pltpu.VMEM((8, 128), jnp.float32),
            pltpu.VMEM((8, 128), jnp.float32),
            pltpu.VMEM((8, 128), jnp.float32),
            pltpu.VMEM((4, L), jnp.float32),
            pltpu.VMEM((4, L), jnp.float32),
            pltpu.VMEM((2, L), jnp.float32),
            pltpu.VMEM((2, 2, L), jnp.int32),
            pltpu.VMEM_SHARED((NS, 8, 128), jnp.float32),
            pltpu.SemaphoreType.DMA,
        ],
    )(preds_r, tgt_r, jnp.asarray(_PAIR_TAB, jnp.int32))
    agg = jnp.concatenate([out[0, 0, :BPC], out[1, 0, :BPC]])
    dis = jnp.concatenate([out[0, 1, :BPC], out[1, 1, :BPC]])
    return agg, dis


# speedup vs baseline: 10.4870x; 1.7396x over previous
import functools

import jax
import jax.numpy as jnp
from jax import lax
from jax.experimental import pallas as pl
from jax.experimental.pallas import tpu as pltpu
from jax.experimental.pallas import tpu_sc as plsc

NC = 2
NS = 16
L = 16
NB = 8
W = 512
NPIX = 512 * 512
BPC = NB // NC
P = NPIX // NS
ROWS = P // W
NIT = P // L

_DELTA_AGG = 0.5
_DELTA_DIS = 3.0

_PAIRS = [(a, b) for a in range(1, 8) for b in range(a + 1, 8)]
_PAIRS = _PAIRS + [(1, 1)] * (32 - len(_PAIRS))
_PAIR_TAB = [[[p[0] for p in _PAIRS[k * 16:(k + 1) * 16]],
              [p[1] for p in _PAIRS[k * 16:(k + 1) * 16]]] for k in range(2)]


def _recip16(x):
    i = jnp.int32(0x7EF477D5) - plsc.bitcast(x, jnp.int32)
    y = plsc.bitcast(i, jnp.float32)
    for _ in range(2):
        y = y * (2.0 - x * y)
    return y


def _sqrt16(s, iters=2):
    i = plsc.bitcast(s, jnp.int32)
    i = jnp.int32(0x5F3759DF) - (i >> 1)
    y = plsc.bitcast(i, jnp.float32)
    for _ in range(iters):
        y = y * (1.5 - 0.5 * s * y * y)
    return s * y


_LOGC = (-2.242481818574011, 4.911042808768637, -5.126667255636585,
         3.9326333882277567, -2.020202093852692, 0.6590148821973966,
         -0.12345843186233507, 0.010119082927734654)


def _log16(x):
    b = plsc.bitcast(x, jnp.int32)
    e = ((b >> 23) - 127).astype(jnp.float32)
    m = plsc.bitcast((b & jnp.int32(0x007FFFFF)) | jnp.int32(0x3F800000),
                     jnp.float32)
    acc = jnp.full((L,), _LOGC[7], jnp.float32)
    for c in _LOGC[6::-1]:
        acc = acc * m + c
    return e * 0.6931471805599453 + acc


def _sc_body(preds_hbm, tgt_hbm, pairs_hbm, out_hbm,
             sv_v, tt_v, tk_v, tab_v, tmp_v, acc_v, gtab_v,
             stat_v, outbuf_v, pairs_v, shared_v, sem):
    cid = lax.axis_index("c")
    sid = lax.axis_index("s")
    lane = lax.iota(jnp.int32, L)
    zeros = jnp.zeros((L,), jnp.float32)
    ones = jnp.full((L,), 1.0, jnp.float32)

    pltpu.sync_copy(pairs_hbm, pairs_v)

    @pl.when(sid == 0)
    def _():
        for r in range(2):
            outbuf_v[r] = zeros

    def reduce_tables(r0, nr):
        pltpu.sync_copy(shared_v.at[0, pl.ds(r0, nr)], acc_v.at[pl.ds(r0, nr)])

        def rbody(s, _):
            pltpu.sync_copy(shared_v.at[s, pl.ds(r0, nr)],
                            tmp_v.at[pl.ds(0, nr)])
            for r in range(nr):
                for j in range(128 // L):
                    ds = pl.ds(j * L, L)
                    acc_v[r0 + r, ds] = acc_v[r0 + r, ds] + tmp_v[r, ds]
            return 0

        lax.fori_loop(1, NS, rbody, 0)

    rbase = sid * ROWS

    def strip_copies(b, make):
        mk = pltpu.make_async_copy if make else pltpu.async_copy
        cps = []
        for ch in range(4):
            cps.append(mk(preds_hbm.at[b, 2 + ch, pl.ds(rbase, ROWS), :],
                          sv_v.at[ch], sem))
        cps.append(mk(tgt_hbm.at[b, 0, pl.ds(rbase, ROWS), :], tt_v, sem))
        cps.append(mk(tgt_hbm.at[b, 1, pl.ds(rbase, ROWS), :], tk_v, sem))
        return cps

    strip_copies(cid * BPC, make=False)

    def batch_body(bl, _):
        b = cid * BPC + bl

        for cp in strip_copies(b, make=True):
            cp.wait()

        for r in range(8):
            for j in range(128 // L):
                tab_v[r, pl.ds(j * L, L)] = zeros

        @plsc.parallel_loop(0, NIT, 1, unroll=8)
        def _(i):
            r = i >> 5
            c = (i & 31) << 4
            idx = (tk_v[r, pl.ds(c, L)] << 4) | lane
            plsc.addupdate_scatter(tab_v.at[4], [idx], ones)
            for ch in range(4):
                plsc.addupdate_scatter(tab_v.at[ch], [idx],
                                       sv_v[ch, r, pl.ds(c, L)])

        pltpu.sync_copy(tab_v.at[pl.ds(0, 5)], shared_v.at[sid, pl.ds(0, 5)])
        plsc.subcore_barrier()
        reduce_tables(0, 5)
        ckvec = zeros
        gv = [zeros, zeros, zeros, zeros]
        for i in range(8):
            sel = lane == i
            ckvec = jnp.where(sel, jnp.sum(acc_v[4, pl.ds(i * L, L)]), ckvec)
            for ch in range(4):
                gv[ch] = jnp.where(sel, jnp.sum(acc_v[ch, pl.ds(i * L, L)]),
                                   gv[ch])
        stat_v[0] = ckvec
        inv = _recip16(jnp.maximum(ckvec, 1.0))
        for ch in range(4):
            gtab_v[ch] = gv[ch] * inv
        plsc.subcore_barrier()

        @plsc.parallel_loop(0, NIT, 1, unroll=6)
        def _(i):
            r = i >> 5
            c = (i & 31) << 4
            tt16 = tt_v[r, pl.ds(c, L)]
            s = zeros
            for ch in range(4):
                d = sv_v[ch, r, pl.ds(c, L)] - plsc.load_gather(
                    gtab_v.at[ch], [tt16])
                s = s + d * d
            s = jnp.maximum(s, 0.25)
            lg = _log16(s - _sqrt16(s, iters=1) + 1.25)
            idx = (tt16 << 4) | lane
            plsc.addupdate_scatter(tab_v.at[5], [idx], lg)
            plsc.addupdate_scatter(tab_v.at[6], [idx], ones)

        @pl.when(bl < BPC - 1)
        def _():
            strip_copies(b + 1, make=False)

        pltpu.sync_copy(tab_v.at[pl.ds(5, 2)], shared_v.at[sid, pl.ds(5, 2)])
        plsc.subcore_barrier()

        @pl.when(sid == 0)
        def _():
            reduce_tables(5, 2)
            lsum = zeros
            ctv = zeros
            for i in range(1, 8):
                sel = lane == i
                lsum = jnp.where(sel, jnp.sum(acc_v[5, pl.ds(i * L, L)]),
                                 lsum)
                ctv = jnp.where(sel, jnp.sum(acc_v[6, pl.ds(i * L, L)]), ctv)
            ckv = stat_v[0]
            lanemask = (lane >= 1) & (lane < 8)
            validf = jnp.where((ctv > 0.0) & (ckv > 0.0) & lanemask, 1.0, 0.0)
            stat_v[3] = validf
            m = zeros + jnp.sum(validf)
            inst = lsum * _recip16(jnp.maximum(ctv, 1.0))
            aggsum = jnp.sum(validf * inst)
            agg_b = jnp.where(m > 0.0,
                              aggsum * _recip16(jnp.maximum(m, 1.0)), 0.0)
            dsum = 0.0
            for k in range(2):
                pa = pairs_v[k, 0]
                pb = pairs_v[k, 1]
                pm = jnp.where(pa < pb, 1.0, 0.0)
                s = zeros
                for ch in range(4):
                    dg = (plsc.load_gather(gtab_v.at[ch], [pa])
                          - plsc.load_gather(gtab_v.at[ch], [pb]))
                    s = s + dg * dg
                nrm = _sqrt16(jnp.maximum(s, 1e-12))
                dd = jnp.maximum(_DELTA_DIS - nrm, 0.0)
                va = plsc.load_gather(stat_v.at[3], [pa])
                vb = plsc.load_gather(stat_v.at[3], [pb])
                dsum = dsum + jnp.sum(_log16(dd * dd + 1.0) * va * vb * pm)
            denom = jnp.maximum(m * (m - 1.0), 1.0)
            dis_b = jnp.where(m > 1.0, dsum * _recip16(denom), 0.0)
            outbuf_v[0] = jnp.where(lane == bl, agg_b, outbuf_v[0])
            outbuf_v[1] = jnp.where(lane == bl, dis_b, outbuf_v[1])
        plsc.subcore_barrier()
        return 0

    lax.fori_loop(0, BPC, batch_body, 0)

    @pl.when(sid == 0)
    def _():
        pltpu.sync_copy(outbuf_v, out_hbm.at[cid])


@jax.jit
def kernel(preds, targets):
    mesh = plsc.VectorSubcoreMesh(core_axis_name="c", subcore_axis_name="s",
                                  num_cores=NC, num_subcores=NS)
    out = pl.kernel(
        _sc_body,
        out_type=jax.ShapeDtypeStruct((NC, 2, L), jnp.float32),
        mesh=mesh,
        compiler_params=pltpu.CompilerParams(needs_layout_passes=False,
                                             use_tc_tiling_on_sc=True),
        scratch_types=[
            pltpu.VMEM((4, ROWS, W), jnp.float32),
            pltpu.VMEM((ROWS, W), jnp.int32),
            pltpu.VMEM((ROWS, W), jnp.int32),
            pltpu.VMEM((8, 128), jnp.float32),
            pltpu.VMEM((8, 128), jnp.float32),
            pltpu.VMEM((8, 128), jnp.float32),
            pltpu.VMEM((4, L), jnp.float32),
            pltpu.VMEM((4, L), jnp.float32),
            pltpu.VMEM((2, L), jnp.float32),
            pltpu.VMEM((2, 2, L), jnp.int32),
            pltpu.VMEM_SHARED((NS, 8, 128), jnp.float32),
            pltpu.SemaphoreType.DMA,
        ],
    )(preds, targets, jnp.asarray(_PAIR_TAB, jnp.int32))
    agg = jnp.concatenate([out[0, 0, :BPC], out[1, 0, :BPC]])
    dis = jnp.concatenate([out[0, 1, :BPC], out[1, 1, :BPC]])
    return agg, dis


# speedup vs baseline: 10.6625x; 1.0167x over previous
import functools

import jax
import jax.numpy as jnp
from jax import lax
from jax.experimental import pallas as pl
from jax.experimental.pallas import tpu as pltpu
from jax.experimental.pallas import tpu_sc as plsc

NC = 2
NS = 16
L = 16
NB = 8
W = 512
NPIX = 512 * 512
BPC = NB // NC
P = NPIX // NS
ROWS = P // W
NIT = P // L

_DELTA_AGG = 0.5
_DELTA_DIS = 3.0

_PAIRS = [(a, b) for a in range(1, 8) for b in range(a + 1, 8)]
_PAIRS = _PAIRS + [(1, 1)] * (32 - len(_PAIRS))
_PAIR_TAB = [[[p[0] for p in _PAIRS[k * 16:(k + 1) * 16]],
              [p[1] for p in _PAIRS[k * 16:(k + 1) * 16]]] for k in range(2)]


def _recip16(x):
    i = jnp.int32(0x7EF477D5) - plsc.bitcast(x, jnp.int32)
    y = plsc.bitcast(i, jnp.float32)
    for _ in range(2):
        y = y * (2.0 - x * y)
    return y


def _sqrt16(s, iters=2):
    i = plsc.bitcast(s, jnp.int32)
    i = jnp.int32(0x5F3759DF) - (i >> 1)
    y = plsc.bitcast(i, jnp.float32)
    for _ in range(iters):
        y = y * (1.5 - 0.5 * s * y * y)
    return s * y


_LOGC = (-1.9316715417209647, 3.498227901209959, -2.420812563219248,
         1.1048082361995168, -0.2806325404497544, 0.030102625011692218)


def _log16(x):
    b = plsc.bitcast(x, jnp.int32)
    e = ((b >> 23) - 127).astype(jnp.float32)
    m = plsc.bitcast((b & jnp.int32(0x007FFFFF)) | jnp.int32(0x3F800000),
                     jnp.float32)
    acc = jnp.full((L,), _LOGC[-1], jnp.float32)
    for c in _LOGC[-2::-1]:
        acc = acc * m + c
    return e * 0.6931471805599453 + acc


def _sc_body(preds_hbm, tgt_hbm, pairs_hbm, out_hbm,
             sv_v, tt_v, tk_v, tab_v, tmp_v, acc_v, gtab_v,
             stat_v, outbuf_v, pairs_v, shared_v, sem):
    cid = lax.axis_index("c")
    sid = lax.axis_index("s")
    lane = lax.iota(jnp.int32, L)
    zeros = jnp.zeros((L,), jnp.float32)
    ones = jnp.full((L,), 1.0, jnp.float32)

    pltpu.sync_copy(pairs_hbm, pairs_v)

    @pl.when(sid == 0)
    def _():
        for r in range(2):
            outbuf_v[r] = zeros

    def reduce_tables(r0, nr):
        pltpu.sync_copy(shared_v.at[0, pl.ds(r0, nr)], acc_v.at[pl.ds(r0, nr)])

        def rbody(s, _):
            pltpu.sync_copy(shared_v.at[s, pl.ds(r0, nr)],
                            tmp_v.at[pl.ds(0, nr)])
            for r in range(nr):
                for j in range(128 // L):
                    ds = pl.ds(j * L, L)
                    acc_v[r0 + r, ds] = acc_v[r0 + r, ds] + tmp_v[r, ds]
            return 0

        lax.fori_loop(1, NS, rbody, 0)

    rbase = sid * ROWS

    def strip_copies(b, make):
        mk = pltpu.make_async_copy if make else pltpu.async_copy
        cps = []
        for ch in range(4):
            cps.append(mk(preds_hbm.at[b, 2 + ch, pl.ds(rbase, ROWS), :],
                          sv_v.at[ch], sem))
        cps.append(mk(tgt_hbm.at[b, 0, pl.ds(rbase, ROWS), :], tt_v, sem))
        cps.append(mk(tgt_hbm.at[b, 1, pl.ds(rbase, ROWS), :], tk_v, sem))
        return cps

    strip_copies(cid * BPC, make=False)

    def batch_body(bl, _):
        b = cid * BPC + bl

        for cp in strip_copies(b, make=True):
            cp.wait()

        for r in range(8):
            for j in range(128 // L):
                tab_v[r, pl.ds(j * L, L)] = zeros

        @plsc.parallel_loop(0, NIT, 1, unroll=8)
        def _(i):
            r = i >> 5
            c = (i & 31) << 4
            idx = (tk_v[r, pl.ds(c, L)] << 4) | lane
            plsc.addupdate_scatter(tab_v.at[4], [idx], ones)
            for ch in range(4):
                plsc.addupdate_scatter(tab_v.at[ch], [idx],
                                       sv_v[ch, r, pl.ds(c, L)])

        pltpu.sync_copy(tab_v.at[pl.ds(0, 5)], shared_v.at[sid, pl.ds(0, 5)])
        plsc.subcore_barrier()
        reduce_tables(0, 5)
        ckvec = zeros
        gv = [zeros, zeros, zeros, zeros]
        for i in range(8):
            sel = lane == i
            ckvec = jnp.where(sel, jnp.sum(acc_v[4, pl.ds(i * L, L)]), ckvec)
            for ch in range(4):
                gv[ch] = jnp.where(sel, jnp.sum(acc_v[ch, pl.ds(i * L, L)]),
                                   gv[ch])
        stat_v[0] = ckvec
        inv = _recip16(jnp.maximum(ckvec, 1.0))
        for ch in range(4):
            gtab_v[ch] = gv[ch] * inv
        plsc.subcore_barrier()

        @plsc.parallel_loop(0, NIT, 1, unroll=6)
        def _(i):
            r = i >> 5
            c = (i & 31) << 4
            tt16 = tt_v[r, pl.ds(c, L)]
            s = zeros
            for ch in range(4):
                d = sv_v[ch, r, pl.ds(c, L)] - plsc.load_gather(
                    gtab_v.at[ch], [tt16])
                s = s + d * d
            s = jnp.maximum(s, 0.25)
            lg = _log16(s - _sqrt16(s, iters=1) + 1.25)
            idx = (tt16 << 4) | lane
            plsc.addupdate_scatter(tab_v.at[5], [idx], lg)
            plsc.addupdate_scatter(tab_v.at[6], [idx], ones)

        @pl.when(bl < BPC - 1)
        def _():
            strip_copies(b + 1, make=False)

        pltpu.sync_copy(tab_v.at[pl.ds(5, 2)], shared_v.at[sid, pl.ds(5, 2)])
        plsc.subcore_barrier()

        @pl.when(sid == 0)
        def _():
            reduce_tables(5, 2)
            lsum = zeros
            ctv = zeros
            for i in range(1, 8):
                sel = lane == i
                lsum = jnp.where(sel, jnp.sum(acc_v[5, pl.ds(i * L, L)]),
                                 lsum)
                ctv = jnp.where(sel, jnp.sum(acc_v[6, pl.ds(i * L, L)]), ctv)
            ckv = stat_v[0]
            lanemask = (lane >= 1) & (lane < 8)
            validf = jnp.where((ctv > 0.0) & (ckv > 0.0) & lanemask, 1.0, 0.0)
            stat_v[3] = validf
            m = zeros + jnp.sum(validf)
            inst = lsum * _recip16(jnp.maximum(ctv, 1.0))
            aggsum = jnp.sum(validf * inst)
            agg_b = jnp.where(m > 0.0,
                              aggsum * _recip16(jnp.maximum(m, 1.0)), 0.0)
            dsum = 0.0
            for k in range(2):
                pa = pairs_v[k, 0]
                pb = pairs_v[k, 1]
                pm = jnp.where(pa < pb, 1.0, 0.0)
                s = zeros
                for ch in range(4):
                    dg = (plsc.load_gather(gtab_v.at[ch], [pa])
                          - plsc.load_gather(gtab_v.at[ch], [pb]))
                    s = s + dg * dg
                nrm = _sqrt16(jnp.maximum(s, 1e-12))
                dd = jnp.maximum(_DELTA_DIS - nrm, 0.0)
                va = plsc.load_gather(stat_v.at[3], [pa])
                vb = plsc.load_gather(stat_v.at[3], [pb])
                dsum = dsum + jnp.sum(_log16(dd * dd + 1.0) * va * vb * pm)
            denom = jnp.maximum(m * (m - 1.0), 1.0)
            dis_b = jnp.where(m > 1.0, dsum * _recip16(denom), 0.0)
            outbuf_v[0] = jnp.where(lane == bl, agg_b, outbuf_v[0])
            outbuf_v[1] = jnp.where(lane == bl, dis_b, outbuf_v[1])
        plsc.subcore_barrier()
        return 0

    lax.fori_loop(0, BPC, batch_body, 0)

    @pl.when(sid == 0)
    def _():
        pltpu.sync_copy(outbuf_v, out_hbm.at[cid])


@jax.jit
def kernel(preds, targets):
    mesh = plsc.VectorSubcoreMesh(core_axis_name="c", subcore_axis_name="s",
                                  num_cores=NC, num_subcores=NS)
    out = pl.kernel(
        _sc_body,
        out_type=jax.ShapeDtypeStruct((NC, 2, L), jnp.float32),
        mesh=mesh,
        compiler_params=pltpu.CompilerParams(needs_layout_passes=False,
                                             use_tc_tiling_on_sc=True),
        scratch_types=[
            pltpu.VMEM((4, ROWS, W), jnp.float32),
            pltpu.VMEM((ROWS, W), jnp.int32),
            pltpu.VMEM((ROWS, W), jnp.int32),
            pltpu.VMEM((8, 128), jnp.float32),
            pltpu.VMEM((8, 128), jnp.float32),
            pltpu.VMEM((8, 128), jnp.float32),
            pltpu.VMEM((4, L), jnp.float32),
            pltpu.VMEM((4, L), jnp.float32),
            pltpu.VMEM((2, L), jnp.float32),
            pltpu.VMEM((2, 2, L), jnp.int32),
            pltpu.VMEM_SHARED((NS, 8, 128), jnp.float32),
            pltpu.SemaphoreType.DMA,
        ],
    )(preds, targets, jnp.asarray(_PAIR_TAB, jnp.int32))
    agg = jnp.concatenate([out[0, 0, :BPC], out[1, 0, :BPC]])
    dis = jnp.concatenate([out[0, 1, :BPC], out[1, 1, :BPC]])
    return agg, dis
